# baseline (speedup 1.0000x reference)
import functools

import jax
import jax.numpy as jnp
from jax import lax
from jax.experimental import pallas as pl
from jax.experimental.pallas import tpu as pltpu
from jax.experimental.pallas import tpu_sc as plsc

N_NODES = 10000
D = 128

NC = 2
NS = 16
NW = NC * NS

C = 128
NH = 2
K2 = 40
K = NH * K2
NBUF = 2
EDGES_PAD = NW * K * C

ACC_ROWS = 10240
DUMMY_ROW = N_NODES

@functools.cache
def _build_sc_message_pass():
    mesh = plsc.VectorSubcoreMesh(core_axis_name="c", subcore_axis_name="s")
    return pl.kernel(
        _sc_message_pass_body,
        out_type=jax.ShapeDtypeStruct((NC, N_NODES, D), jnp.float32),
        mesh=mesh,
        scratch_types=[
            pltpu.VMEM((K2, C), jnp.int32),
            pltpu.VMEM((K2, C), jnp.int32),
        ] + [pltpu.VMEM((C, D), jnp.float32) for _ in range(NBUF)]
          + [pltpu.SemaphoreType.DMA((NBUF,)),
             pltpu.SemaphoreType.DMA((NBUF,)),
             pltpu.VMEM_SHARED((ACC_ROWS, D), jnp.float32)],
    )


def _sc_message_pass_body(feat_hbm, src_hbm, dst_hbm, out_hbm,
                          src_v, dst_v, *rest):
    bufs = rest[:NBUF]
    gsem, ssem, acc_sh = rest[NBUF], rest[NBUF + 1], rest[NBUF + 2]
    cid = lax.axis_index("c")
    sid = lax.axis_index("s")
    wid = cid * NS + sid

    zbuf = bufs[0]

    @pl.loop(0, C)
    def _(r):
        @pl.loop(0, D, step=16)
        def _(c):
            zbuf.at[pl.ds(r, 1), pl.ds(c, 16)][...] = jnp.zeros(
                (1, 16), jnp.float32)

    @pl.loop(0, ACC_ROWS // (NS * C))
    def _(k):
        pltpu.sync_copy(zbuf, acc_sh.at[pl.ds(sid * (ACC_ROWS // NS) + k * C, C)])

    plsc.subcore_barrier()

    def start_gather(j, b):
        pltpu.async_copy(feat_hbm.at[src_v.at[j]], bufs[b], gsem.at[b])

    def wait_gather(j, b):
        pltpu.make_async_copy(feat_hbm.at[src_v.at[j]], bufs[b], gsem.at[b]).wait()

    def start_scatter(j, b):
        pltpu.async_copy(bufs[b], acc_sh.at[dst_v.at[j]], ssem.at[b], add=True)

    def wait_scatter(j, b):
        pltpu.make_async_copy(bufs[b], acc_sh.at[dst_v.at[j]], ssem.at[b]).wait()

    for half in range(NH):
        pltpu.sync_copy(src_hbm.at[wid, half], src_v)
        pltpu.sync_copy(dst_hbm.at[wid, half], dst_v)

        for b in range(NBUF):
            start_gather(b, b)

        @pl.loop(0, K2 - NBUF, step=NBUF)
        def _(jj):
            for b in range(NBUF):
                wait_gather(jj + b, b)
                start_scatter(jj + b, b)
            for b in range(NBUF):
                wait_scatter(jj + b, b)
                start_gather(jj + NBUF + b, b)

        for b in range(NBUF):
            j = K2 - NBUF + b
            wait_gather(j, b)
            start_scatter(j, b)
        for b in range(NBUF):
            j = K2 - NBUF + b
            wait_scatter(j, b)

    plsc.subcore_barrier()

    rows_per = 624
    pltpu.sync_copy(acc_sh.at[pl.ds(sid * rows_per, rows_per)],
                    out_hbm.at[cid, pl.ds(sid * rows_per, rows_per)])

    @pl.when(sid == 0)
    def _():
        pltpu.sync_copy(acc_sh.at[pl.ds(NS * rows_per, N_NODES - NS * rows_per)],
                        out_hbm.at[cid, pl.ds(NS * rows_per, N_NODES - NS * rows_per)])


def _tc_linear_body(p_ref, w_ref, b_ref, o_ref):
    h = p_ref[0] + p_ref[1]
    y = lax.dot_general(
        h, w_ref[...],
        dimension_numbers=(((1,), (1,)), ((), ())),
        precision=lax.Precision.HIGHEST,
        preferred_element_type=jnp.float32,
    )
    o_ref[...] = jnp.maximum(y + b_ref[...], 0.0)


def kernel(feature, edge_index, W, b):
    n_edges = edge_index.shape[1]
    pad = EDGES_PAD - n_edges
    src = jnp.concatenate([edge_index[0], jnp.zeros((pad,), jnp.int32)])
    dst = jnp.concatenate(
        [edge_index[1], jnp.full((pad,), DUMMY_ROW, jnp.int32)])
    src3 = src.reshape(NW, NH, K2, C)
    dst3 = dst.reshape(NW, NH, K2, C)

    partials = _build_sc_message_pass()(feature, src3, dst3)

    rows_blk = 1000
    grid = (N_NODES // rows_blk,)
    out = pl.pallas_call(
        _tc_linear_body,
        grid=grid,
        in_specs=[
            pl.BlockSpec((NC, rows_blk, D), lambda i: (0, i, 0)),
            pl.BlockSpec((D, D), lambda i: (0, 0)),
            pl.BlockSpec((1, D), lambda i: (0, 0)),
        ],
        out_specs=pl.BlockSpec((rows_blk, D), lambda i: (i, 0)),
        out_shape=jax.ShapeDtypeStruct((N_NODES, D), jnp.float32),
    )(partials, W, b.reshape(1, D))
    return out


# speedup vs baseline: 1.4351x; 1.4351x over previous
import functools

import jax
import jax.numpy as jnp
from jax import lax
from jax.experimental import pallas as pl
from jax.experimental.pallas import tpu as pltpu
from jax.experimental.pallas import tpu_sc as plsc

N_NODES = 10000
D = 128

NC = 2
NS = 16
NW = NC * NS

C = 128
K = 79
EDGES_PAD = NW * K * C

ACC_ROWS = 10240
DUMMY_ROW = N_NODES


@functools.cache
def _build_sc_message_pass():
    mesh = plsc.VectorSubcoreMesh(core_axis_name="c", subcore_axis_name="s")
    return pl.kernel(
        _sc_message_pass_body,
        out_type=jax.ShapeDtypeStruct((NC, N_NODES, D), jnp.float32),
        mesh=mesh,
        scratch_types=[
            pltpu.VMEM((K, C), jnp.int32),
            pltpu.VMEM((K, C), jnp.int32),
            pltpu.VMEM((C, D), jnp.float32),
            pltpu.VMEM_SHARED((ACC_ROWS, D), jnp.float32),
        ],
    )


def _sc_message_pass_body(feat_hbm, src_hbm, dst_hbm, out_hbm,
                          src_v, dst_v, rows_v, acc_sh):
    cid = lax.axis_index("c")
    sid = lax.axis_index("s")
    wid = cid * NS + sid

    @pl.loop(0, C)
    def _(r):
        @pl.loop(0, D, step=16)
        def _(c):
            rows_v.at[pl.ds(r, 1), pl.ds(c, 16)][...] = jnp.zeros(
                (1, 16), jnp.float32)

    @pl.loop(0, ACC_ROWS // (NS * C))
    def _(k):
        pltpu.sync_copy(rows_v, acc_sh.at[pl.ds(sid * (ACC_ROWS // NS) + k * C, C)])

    plsc.subcore_barrier()

    pltpu.sync_copy(src_hbm.at[wid], src_v)
    pltpu.sync_copy(dst_hbm.at[wid], dst_v)

    @pl.loop(0, K)
    def _(j):
        pltpu.sync_copy(feat_hbm.at[src_v.at[j]], rows_v)
        pltpu.sync_copy(rows_v, acc_sh.at[dst_v.at[j]], add=True)

    plsc.subcore_barrier()

    rows_per = 624
    pltpu.sync_copy(acc_sh.at[pl.ds(sid * rows_per, rows_per)],
                    out_hbm.at[cid, pl.ds(sid * rows_per, rows_per)])

    @pl.when(sid == 0)
    def _():
        pltpu.sync_copy(acc_sh.at[pl.ds(NS * rows_per, N_NODES - NS * rows_per)],
                        out_hbm.at[cid, pl.ds(NS * rows_per, N_NODES - NS * rows_per)])


def _tc_linear_body(p_ref, w_ref, b_ref, o_ref):
    h = p_ref[0] + p_ref[1]
    y = lax.dot_general(
        h, w_ref[...],
        dimension_numbers=(((1,), (1,)), ((), ())),
        precision=lax.Precision.HIGHEST,
        preferred_element_type=jnp.float32,
    )
    o_ref[...] = jnp.maximum(y + b_ref[...], 0.0)


def kernel(feature, edge_index, W, b):
    n_edges = edge_index.shape[1]
    pad = EDGES_PAD - n_edges
    src = jnp.concatenate([edge_index[0], jnp.zeros((pad,), jnp.int32)])
    dst = jnp.concatenate(
        [edge_index[1], jnp.full((pad,), DUMMY_ROW, jnp.int32)])
    src3 = src.reshape(NW, K, C)
    dst3 = dst.reshape(NW, K, C)

    partials = _build_sc_message_pass()(feature, src3, dst3)

    rows_blk = 1000
    grid = (N_NODES // rows_blk,)
    out = pl.pallas_call(
        _tc_linear_body,
        grid=grid,
        in_specs=[
            pl.BlockSpec((NC, rows_blk, D), lambda i: (0, i, 0)),
            pl.BlockSpec((D, D), lambda i: (0, 0)),
            pl.BlockSpec((1, D), lambda i: (0, 0)),
        ],
        out_specs=pl.BlockSpec((rows_blk, D), lambda i: (i, 0)),
        out_shape=jax.ShapeDtypeStruct((N_NODES, D), jnp.float32),
    )(partials, W, b.reshape(1, D))
    return out


# speedup vs baseline: 3.4400x; 2.3971x over previous
import functools

import jax
import jax.numpy as jnp
from jax import lax
from jax.experimental import pallas as pl
from jax.experimental.pallas import tpu as pltpu
from jax.experimental.pallas import tpu_sc as plsc

N_NODES = 10000
D = 128

NC = 2
NS = 16
NW = NC * NS

C = 128
K = 84
NBUF = 3
NIB = 6
EDGES_PAD = NW * K * C

ACC_ROWS = 10016
DUMMY_ROW = N_NODES


@functools.cache
def _build_sc_message_pass():
    mesh = plsc.VectorSubcoreMesh(core_axis_name="c", subcore_axis_name="s")
    return pl.kernel(
        _sc_message_pass_body,
        out_type=jax.ShapeDtypeStruct((NC, N_NODES, D), jnp.float32),
        mesh=mesh,
        scratch_types=(
            [pltpu.VMEM((2, C), jnp.int32) for _ in range(NIB)]
            + [pltpu.VMEM((C, D), jnp.float32) for _ in range(NBUF)]
            + [pltpu.SemaphoreType.DMA((NIB,)),
               pltpu.SemaphoreType.DMA((NBUF,)),
               pltpu.SemaphoreType.DMA((NBUF,)),
               pltpu.VMEM_SHARED((ACC_ROWS, D), jnp.float32)]
        ),
    )


def _sc_message_pass_body(feat_hbm, idx_hbm, out_hbm, *rest):
    slots = rest[:NIB]
    bufs = rest[NIB:NIB + NBUF]
    isem, gsem, ssem, acc_sh = rest[NIB + NBUF:]
    cid = lax.axis_index("c")
    sid = lax.axis_index("s")
    wid = cid * NS + sid

    zbuf = bufs[0]

    @pl.loop(0, C)
    def _(r):
        @pl.loop(0, D, step=16)
        def _(c):
            zbuf.at[pl.ds(r, 1), pl.ds(c, 16)][...] = jnp.zeros(
                (1, 16), jnp.float32)

    span = ACC_ROWS // NS
    for k in range(4):
        pltpu.sync_copy(zbuf, acc_sh.at[pl.ds(sid * span + k * C, C)])
    pltpu.sync_copy(zbuf.at[pl.ds(0, span - 4 * C)],
                    acc_sh.at[pl.ds(sid * span + 4 * C, span - 4 * C)])

    plsc.subcore_barrier()

    def start_idx(j, s):
        pltpu.async_copy(idx_hbm.at[wid, j], slots[s], isem.at[s])

    def wait_idx(j, s):
        pltpu.make_async_copy(idx_hbm.at[wid, j], slots[s], isem.at[s]).wait()

    def start_gather(j, s, b):
        pltpu.async_copy(feat_hbm.at[slots[s].at[0]], bufs[b], gsem.at[b])

    def wait_gather(j, s, b):
        pltpu.make_async_copy(feat_hbm.at[slots[s].at[0]], bufs[b],
                              gsem.at[b]).wait()

    def start_scatter(j, s, b):
        pltpu.async_copy(bufs[b], acc_sh.at[slots[s].at[1]], ssem.at[b],
                         add=True)

    def wait_scatter(j, s, b):
        pltpu.make_async_copy(bufs[b], acc_sh.at[slots[s].at[1]],
                              ssem.at[b]).wait()

    for s in range(NIB):
        start_idx(s, s)
    for b in range(NBUF):
        wait_idx(b, b)
        start_gather(b, b, b)

    def half_phase(jj, h, nxt_gather, nxt_idx):
        for b in range(NBUF):
            j = jj + h + b
            s = (h + b) % NIB
            wait_gather(j, s, b)
            start_scatter(j, s, b)
        for b in range(NBUF):
            j = jj + h + b
            s = (h + b) % NIB
            wait_scatter(j, s, b)
            if nxt_gather:
                s3 = (h + b + NBUF) % NIB
                wait_idx(j + NBUF, s3)
                start_gather(j + NBUF, s3, b)
            if nxt_idx:
                start_idx(j + NIB, s)

    @pl.loop(0, K - NIB, step=NIB)
    def _(jj):
        half_phase(jj, 0, True, True)
        half_phase(jj, NBUF, True, True)

    half_phase(K - NIB, 0, True, False)
    half_phase(K - NIB, NBUF, False, False)

    plsc.subcore_barrier()

    rows_per = 624
    pltpu.sync_copy(acc_sh.at[pl.ds(sid * rows_per, rows_per)],
                    out_hbm.at[cid, pl.ds(sid * rows_per, rows_per)])

    @pl.when(sid == 0)
    def _():
        pltpu.sync_copy(acc_sh.at[pl.ds(NS * rows_per, N_NODES - NS * rows_per)],
                        out_hbm.at[cid, pl.ds(NS * rows_per, N_NODES - NS * rows_per)])


def _tc_linear_body(p_ref, w_ref, b_ref, o_ref):
    h = p_ref[0] + p_ref[1]
    y = lax.dot_general(
        h, w_ref[...],
        dimension_numbers=(((1,), (1,)), ((), ())),
        precision=lax.Precision.HIGHEST,
        preferred_element_type=jnp.float32,
    )
    o_ref[...] = jnp.maximum(y + b_ref[...], 0.0)


def kernel(feature, edge_index, W, b):
    n_edges = edge_index.shape[1]
    pad = EDGES_PAD - n_edges
    pad_ar = jnp.arange(pad, dtype=jnp.int32)
    src = jnp.concatenate([edge_index[0], pad_ar % N_NODES])
    dst = jnp.concatenate([edge_index[1], DUMMY_ROW + (pad_ar % 16)])
    idx = jnp.stack([src.reshape(NW, K, C), dst.reshape(NW, K, C)], axis=2)

    partials = _build_sc_message_pass()(feature, idx)

    rows_blk = 1000
    grid = (N_NODES // rows_blk,)
    out = pl.pallas_call(
        _tc_linear_body,
        grid=grid,
        in_specs=[
            pl.BlockSpec((NC, rows_blk, D), lambda i: (0, i, 0)),
            pl.BlockSpec((D, D), lambda i: (0, 0)),
            pl.BlockSpec((1, D), lambda i: (0, 0)),
        ],
        out_specs=pl.BlockSpec((rows_blk, D), lambda i: (i, 0)),
        out_shape=jax.ShapeDtypeStruct((N_NODES, D), jnp.float32),
    )(partials, W, b.reshape(1, D))
    return out
